# Initial kernel scaffold; baseline (speedup 1.0000x reference)
import jax
import jax.numpy as jnp
from jax.experimental import pallas as pl
from jax.experimental.pallas import tpu as pltpu
from jax.experimental.pallas import tpu_sc as plsc

NUM_K = 8192
DIM = 32
N_TOKENS = 16384
NT = 128
GW = 256


def _vq_body(x_ref, wt_ref, idx_ref, loss_ref, wtb_ref, b_ref):
    i = pl.program_id(0)

    @pl.when(i == 0)
    def _init():
        wtf = wt_ref[...]
        wtb_ref[...] = wtf.astype(jnp.bfloat16)
        b_ref[...] = jnp.sum(wtf * wtf, axis=0, keepdims=True)
        loss_ref[0, 0] = 0.0

    xt = x_ref[...]
    xb = xt.astype(jnp.bfloat16)
    m = jnp.dot(xb, wtb_ref[...], preferred_element_type=jnp.float32)
    a = jnp.sum(xt * xt, axis=1, keepdims=True)
    t = a + b_ref[...]
    d = t - 2.0 * m
    minval = jnp.min(d, axis=1, keepdims=True)
    iota = jax.lax.broadcasted_iota(jnp.int32, d.shape, 1)
    cand = jnp.where(d == minval, iota, jnp.int32(2**30))
    idx_ref[...] = jnp.min(cand, axis=1, keepdims=True)
    loss_ref[0, 0] += jnp.sum(minval)


def _argmin_and_loss(x_flat, wt):
    grid = (N_TOKENS // NT,)
    return pl.pallas_call(
        _vq_body,
        grid=grid,
        in_specs=[
            pl.BlockSpec((NT, DIM), lambda i: (i, 0)),
            pl.BlockSpec((DIM, NUM_K), lambda i: (0, 0)),
        ],
        out_specs=[
            pl.BlockSpec((NT, 1), lambda i: (i, 0)),
            pl.BlockSpec((1, 1), lambda i: (0, 0)),
        ],
        out_shape=[
            jax.ShapeDtypeStruct((N_TOKENS, 1), jnp.int32),
            jax.ShapeDtypeStruct((1, 1), jnp.float32),
        ],
        scratch_shapes=[
            pltpu.VMEM((DIM, NUM_K), jnp.bfloat16),
            pltpu.VMEM((1, NUM_K), jnp.float32),
        ],
        compiler_params=pltpu.CompilerParams(
            dimension_semantics=("arbitrary",),
        ),
    )(x_flat, wt)


def _sc_gather(W, idx_row):
    mesh = plsc.VectorSubcoreMesh(core_axis_name="c", subcore_axis_name="s")

    @pl.kernel(
        out_type=jax.ShapeDtypeStruct((N_TOKENS, DIM), jnp.float32),
        mesh=mesh,
    )
    def gather_kernel(w_hbm, i_hbm, o_hbm):
        def body(i_vmem, o_vmem):
            pltpu.sync_copy(w_hbm.at[i_vmem.at[0]], o_vmem)

        pltpu.emit_pipeline(
            body,
            grid=(N_TOKENS // GW,),
            in_specs=[pl.BlockSpec((1, GW), index_map=lambda i: (0, i))],
            out_specs=[pl.BlockSpec((GW, DIM), index_map=lambda i: (i, 0))],
            core_axis_name=("c", "s"),
            dimension_semantics=(pltpu.PARALLEL,),
        )(i_hbm, o_hbm)

    return gather_kernel(W, idx_row)


def kernel(x, W):
    B, C, H, Wd = x.shape
    x_flat = jnp.transpose(x, (0, 2, 3, 1)).reshape(-1, DIM)
    wt = W.T

    idx_col, loss_sum = _argmin_and_loss(x_flat, wt)
    idx_flat = idx_col.reshape(-1)

    q_flat = _sc_gather(W, idx_flat.reshape(1, N_TOKENS))

    quantized = q_flat.reshape(B, H, Wd, C)
    out = jnp.transpose(quantized, (0, 3, 1, 2))
    indices = idx_flat.reshape(B, H, Wd)[:, None, :, :]
    mse = loss_sum[0, 0] / jnp.float32(N_TOKENS * DIM)
    loss = mse + 0.25 * mse
    return (out, indices, loss)


# speedup vs baseline: 1.2910x; 1.2910x over previous
import jax
import jax.numpy as jnp
from jax.experimental import pallas as pl
from jax.experimental.pallas import tpu as pltpu
from jax.experimental.pallas import tpu_sc as plsc

NUM_K = 8192
DIM = 32
N_TOKENS = 16384
NT = 128
GW = 256


def _vq_body(x_ref, wt_ref, idx_ref, loss_ref, wtb_ref, b_ref):
    i = pl.program_id(0)

    @pl.when(i == 0)
    def _init():
        wtf = wt_ref[...]
        wtb_ref[...] = wtf.astype(jnp.bfloat16)
        b_ref[...] = jnp.sum(wtf * wtf, axis=0, keepdims=True)
        loss_ref[...] = jnp.zeros((1, 1), jnp.float32)

    xt = x_ref[...]
    xb = xt.astype(jnp.bfloat16)
    m = jnp.dot(xb, wtb_ref[...], preferred_element_type=jnp.float32)
    a = jnp.sum(xt * xt, axis=1, keepdims=True)
    t = a + b_ref[...]
    d = t - 2.0 * m

    half = NUM_K // 2

    def half_argmin(dh, lo):
        mv = jnp.min(dh, axis=1, keepdims=True)
        iota = jax.lax.broadcasted_iota(jnp.int32, dh.shape, 1) + lo
        cand = jnp.where(dh == mv, iota, jnp.int32(2**30))
        return mv, jnp.min(cand, axis=1, keepdims=True)

    v0, i0 = half_argmin(d[:, :half], 0)
    v1, i1 = half_argmin(d[:, half:], half)
    bits = jax.lax.bitcast_convert_type(v0, jnp.int32)
    lsb = jax.lax.shift_right_logical(bits, 16) & jnp.int32(1)
    bits = (bits + jnp.int32(0x7FFF) + lsb) & jnp.int32(-65536)
    carry = jax.lax.bitcast_convert_type(bits, jnp.float32)
    take1 = v1 <= carry
    idx_ref[...] = jnp.where(take1, i1, i0)
    sel = jnp.where(take1, v1, v0)
    loss_ref[...] += jnp.sum(sel).reshape(1, 1)


def _argmin_and_loss(x_flat, wt):
    grid = (N_TOKENS // NT,)
    return pl.pallas_call(
        _vq_body,
        grid=grid,
        in_specs=[
            pl.BlockSpec((NT, DIM), lambda i: (i, 0)),
            pl.BlockSpec((DIM, NUM_K), lambda i: (0, 0)),
        ],
        out_specs=[
            pl.BlockSpec((NT, 1), lambda i: (i, 0)),
            pl.BlockSpec((1, 1), lambda i: (0, 0)),
        ],
        out_shape=[
            jax.ShapeDtypeStruct((N_TOKENS, 1), jnp.int32),
            jax.ShapeDtypeStruct((1, 1), jnp.float32),
        ],
        scratch_shapes=[
            pltpu.VMEM((DIM, NUM_K), jnp.bfloat16),
            pltpu.VMEM((1, NUM_K), jnp.float32),
        ],
        compiler_params=pltpu.CompilerParams(
            dimension_semantics=("arbitrary",),
        ),
    )(x_flat, wt)


GPAD = 128


def _sc_gather(Wp, idx_row):
    mesh = plsc.VectorSubcoreMesh(core_axis_name="c", subcore_axis_name="s")

    @pl.kernel(
        out_type=jax.ShapeDtypeStruct((N_TOKENS, GPAD), jnp.float32),
        mesh=mesh,
    )
    def gather_kernel(w_hbm, i_hbm, o_hbm):
        def body(i_vmem, o_vmem):
            pltpu.sync_copy(w_hbm.at[i_vmem.at[0]], o_vmem)

        pltpu.emit_pipeline(
            body,
            grid=(N_TOKENS // GW,),
            in_specs=[pl.BlockSpec((1, GW), index_map=lambda i: (0, i))],
            out_specs=[pl.BlockSpec((GW, GPAD), index_map=lambda i: (i, 0))],
            core_axis_name=("c", "s"),
            dimension_semantics=(pltpu.PARALLEL,),
        )(i_hbm, o_hbm)

    return gather_kernel(Wp, idx_row)


def kernel(x, W):
    B, C, H, Wd = x.shape
    x_flat = jnp.transpose(x, (0, 2, 3, 1)).reshape(-1, DIM)
    wt = W.T

    idx_col, loss_sum = _argmin_and_loss(x_flat, wt)
    idx_flat = idx_col.reshape(-1)

    Wp = jnp.pad(W, ((0, 0), (0, GPAD - DIM)))
    q_pad = _sc_gather(Wp, idx_flat.reshape(1, N_TOKENS))
    q_flat = q_pad[:, :DIM]

    quantized = q_flat.reshape(B, H, Wd, C)
    out = jnp.transpose(quantized, (0, 3, 1, 2))
    indices = idx_flat.reshape(B, H, Wd)[:, None, :, :]
    mse = loss_sum[0, 0] / jnp.float32(N_TOKENS * DIM)
    loss = mse + 0.25 * mse
    return (out, indices, loss)
